# baseline (speedup 1.0000x reference)
import jax
import jax.numpy as jnp
from jax import lax
from jax.experimental import pallas as pl
from jax.experimental.pallas import tpu as pltpu
from jax.experimental.pallas import tpu_sc as plsc

HASH_N = 1000000
D = 64
B = 16384

_info = plsc.get_sparse_core_info()
NC, NS, L = _info.num_cores, _info.num_subcores, _info.num_lanes
NW = NC * NS
BPW = B // NW
CHUNK = 128
NCHUNK = BPW // CHUNK


def _body(tgt, ctx, tin, tout, out, idx_t, idx_c, rows_t, rows_c, res, sbuf, sem):
    wid = lax.axis_index("s") * NC + lax.axis_index("c")
    base = wid * BPW

    pltpu.sync_copy(tgt.at[pl.ds(base, BPW)], idx_t)
    pltpu.sync_copy(ctx.at[pl.ds(base, BPW)], idx_c)

    def mod_body(j, _):
        sl = pl.ds(j * L, L)
        idx_t[sl] = lax.rem(idx_t[sl], HASH_N)
        idx_c[sl] = lax.rem(idx_c[sl], HASH_N)
        return 0

    lax.fori_loop(0, BPW // L, mod_body, 0)

    copies = []
    for k in range(NCHUNK):
        sl = pl.ds(k * CHUNK, CHUNK)
        copies.append(pltpu.async_copy(tin.at[idx_t.at[sl]], rows_t.at[sl], sem))
        copies.append(pltpu.async_copy(tout.at[idx_c.at[sl]], rows_c.at[sl], sem))
    for c in copies:
        c.wait()

    row_iota = lax.iota(jnp.int32, L)

    def dot_group(g, _):
        rb = g * L
        tot = jnp.zeros((L,), jnp.float32)
        for i in range(L):
            r = rb + i
            acc = rows_t[r, pl.ds(0, L)] * rows_c[r, pl.ds(0, L)]
            for k in range(1, D // L):
                acc = acc + rows_t[r, pl.ds(k * L, L)] * rows_c[r, pl.ds(k * L, L)]
            tot = jnp.where(row_iota == i, jnp.sum(acc), tot)
        res[pl.ds(rb, L)] = tot
        return 0

    lax.fori_loop(0, BPW // L, dot_group, 0)

    pltpu.sync_copy(res, out.at[pl.ds(base, BPW)])


def kernel(target, context, in_embed, out_embed):
    k = pl.kernel(
        _body,
        out_type=jax.ShapeDtypeStruct((B,), jnp.float32),
        mesh=plsc.VectorSubcoreMesh(core_axis_name="c", subcore_axis_name="s"),
        compiler_params=pltpu.CompilerParams(
            needs_layout_passes=False, use_tc_tiling_on_sc=False
        ),
        scratch_types=[
            pltpu.VMEM((BPW,), jnp.int32),
            pltpu.VMEM((BPW,), jnp.int32),
            pltpu.VMEM((BPW, D), jnp.float32),
            pltpu.VMEM((BPW, D), jnp.float32),
            pltpu.VMEM((BPW,), jnp.float32),
            pltpu.VMEM((L * L,), jnp.float32),
            pltpu.SemaphoreType.DMA,
        ],
    )
    return k(target, context, in_embed, out_embed)


# speedup vs baseline: 1.5686x; 1.5686x over previous
import jax
import jax.numpy as jnp
from jax import lax
from jax.experimental import pallas as pl
from jax.experimental.pallas import tpu as pltpu
from jax.experimental.pallas import tpu_sc as plsc

HASH_N = 1000000
D = 64
B = 16384

_info = plsc.get_sparse_core_info()
NC, NS, L = _info.num_cores, _info.num_subcores, _info.num_lanes
NW = NC * NS
BPW = B // NW
CH = 64
NCH = BPW // CH


def _body(tgt, ctx, tin, tout, out, idx_t, idx_c, rows_t, rows_c, res, sem):
    wid = lax.axis_index("s") * NC + lax.axis_index("c")
    base = wid * BPW

    pltpu.sync_copy(tgt.at[pl.ds(base, BPW)], idx_t)
    pltpu.sync_copy(ctx.at[pl.ds(base, BPW)], idx_c)

    def mod_body(j, _):
        sl = pl.ds(j * L, L)
        idx_t[sl] = lax.rem(idx_t[sl], HASH_N)
        idx_c[sl] = lax.rem(idx_c[sl], HASH_N)
        return 0

    lax.fori_loop(0, BPW // L, mod_body, 0)

    row_iota = lax.iota(jnp.int32, L)

    def chunk_body(n, _):
        cb = n * CH
        copies = []
        for q in range(CH // L):
            tvec = idx_t[pl.ds(cb + q * L, L)]
            cvec = idx_c[pl.ds(cb + q * L, L)]
            for i in range(L):
                j = q * L + i
                copies.append(pltpu.async_copy(
                    tin.at[pl.ds(tvec[i], 1)], rows_t.at[pl.ds(j, 1)], sem))
                copies.append(pltpu.async_copy(
                    tout.at[pl.ds(cvec[i], 1)], rows_c.at[pl.ds(j, 1)], sem))
        for c in copies:
            c.wait()

        def dot_group(g, _):
            rb = g * L
            tot = jnp.zeros((L,), jnp.float32)
            for i in range(L):
                r = rb + i
                acc = rows_t[r, pl.ds(0, L)] * rows_c[r, pl.ds(0, L)]
                for k in range(1, D // L):
                    acc = acc + rows_t[r, pl.ds(k * L, L)] * rows_c[r, pl.ds(k * L, L)]
                tot = jnp.where(row_iota == i, jnp.sum(acc), tot)
            res[pl.ds(cb + rb, L)] = tot
            return 0

        lax.fori_loop(0, CH // L, dot_group, 0)
        return 0

    lax.fori_loop(0, NCH, chunk_body, 0)

    pltpu.sync_copy(res, out.at[pl.ds(base, BPW)])


def kernel(target, context, in_embed, out_embed):
    k = pl.kernel(
        _body,
        out_type=jax.ShapeDtypeStruct((B,), jnp.float32),
        mesh=plsc.VectorSubcoreMesh(core_axis_name="c", subcore_axis_name="s"),
        compiler_params=pltpu.CompilerParams(needs_layout_passes=False),
        scratch_types=[
            pltpu.VMEM((BPW,), jnp.int32),
            pltpu.VMEM((BPW,), jnp.int32),
            pltpu.VMEM((CH, D), jnp.float32),
            pltpu.VMEM((CH, D), jnp.float32),
            pltpu.VMEM((BPW,), jnp.float32),
            pltpu.SemaphoreType.DMA,
        ],
    )
    return k(target, context, in_embed, out_embed)
